# baseline (speedup 1.0000x reference)
import jax
import jax.numpy as jnp
from jax import lax
from jax.experimental import pallas as pl
from jax.experimental.pallas import tpu as pltpu
from jax.experimental.pallas import tpu_sc as plsc

_B, _D, _N, _K, _P, _FE = 4, 128, 1024, 16, 16384, 4
_NK = _N * _K
_R = 128
_EPS = 1e-5


def _pre_body(f_ref, w0_ref, idx_ref, src_ref, dst_ref,
              gt_ref, c1_ref, ftt_ref, idxg_ref, qk_ref, ek_ref):
    b = pl.program_id(0)
    x = f_ref[0]
    w0 = w0_ref[...]
    wa = w0[:, :_D]
    wb = w0[:, _D:2 * _D]
    dn = (((0,), (1,)), ((), ()))
    gt_ref[0] = lax.dot_general(x, wb, dn, preferred_element_type=jnp.float32)
    c1_ref[0] = lax.dot_general(x, wa - wb, dn, preferred_element_type=jnp.float32)
    eye = (lax.broadcasted_iota(jnp.int32, (_D, _D), 0) ==
           lax.broadcasted_iota(jnp.int32, (_D, _D), 1)).astype(jnp.float32)
    ftt_ref[0] = lax.dot_general(x, eye, dn, preferred_element_type=jnp.float32)
    idx = idx_ref[0]
    r = lax.broadcasted_iota(jnp.int32, (_R, _R), 0)
    c = lax.broadcasted_iota(jnp.int32, (_R, _R), 1)
    n = (r * _R + c) >> 4
    qk_ref[0] = (n << 10) + idx
    idxg_ref[0] = idx + b * _N
    ek_ref[0] = (src_ref[0] << 10) + dst_ref[0]


def _pre_call(features, W0, idx_r, src_r, dst_r):
    f32, i32 = jnp.float32, jnp.int32
    return pl.pallas_call(
        _pre_body,
        grid=(_B,),
        in_specs=[
            pl.BlockSpec((1, _D, _N), lambda b: (b, 0, 0)),
            pl.BlockSpec((128, 2 * _D + _FE), lambda b: (0, 0)),
            pl.BlockSpec((1, _R, _R), lambda b: (b, 0, 0)),
            pl.BlockSpec((1, _R, _R), lambda b: (b, 0, 0)),
            pl.BlockSpec((1, _R, _R), lambda b: (b, 0, 0)),
        ],
        out_specs=[
            pl.BlockSpec((1, _N, 128), lambda b: (b, 0, 0)),
            pl.BlockSpec((1, _N, 128), lambda b: (b, 0, 0)),
            pl.BlockSpec((1, _N, 128), lambda b: (b, 0, 0)),
            pl.BlockSpec((1, _R, _R), lambda b: (b, 0, 0)),
            pl.BlockSpec((1, _R, _R), lambda b: (b, 0, 0)),
            pl.BlockSpec((1, _R, _R), lambda b: (b, 0, 0)),
        ],
        out_shape=[
            jax.ShapeDtypeStruct((_B, _N, 128), f32),
            jax.ShapeDtypeStruct((_B, _N, 128), f32),
            jax.ShapeDtypeStruct((_B, _N, 128), f32),
            jax.ShapeDtypeStruct((_B, _R, _R), i32),
            jax.ShapeDtypeStruct((_B, _R, _R), i32),
            jax.ShapeDtypeStruct((_B, _R, _R), i32),
        ],
    )(features, W0, idx_r, src_r, dst_r)


def _sc_body(gt2, idxg, qk_h, ek_h, ev_h, gg_out, efq_out,
             ekv, qkv, evv, zlin, ga, gb, gouts, idxv, rowsv,
             sem0, semw, table):
    c = lax.axis_index("c")
    s = lax.axis_index("s")

    z16 = jnp.zeros((16,), jnp.float32)
    for j in range(256):
        zlin[pl.ds(j * 16, 16)] = z16

    zcps = [pltpu.async_copy(zlin, table.at[pl.ds((s * 16 + i) * 4096, 4096)],
                             sem0) for i in range(16)]
    for cp in zcps:
        cp.wait()
    plsc.subcore_barrier()

    wcps = []
    for bi in range(2):
        b = c * 2 + bi
        scps = [pltpu.async_copy(ek_h.at[b, pl.ds(s * 8, 8)], ekv, sem0),
                pltpu.async_copy(qk_h.at[b, pl.ds(s * 8, 8)], qkv, sem0),
                pltpu.async_copy(ev_h.at[b, 0, pl.ds(s * 8, 8)], evv.at[0], sem0),
                pltpu.async_copy(ev_h.at[b, 1, pl.ds(s * 8, 8)], evv.at[1], sem0),
                pltpu.async_copy(ev_h.at[b, 2, pl.ds(s * 8, 8)], evv.at[2], sem0),
                pltpu.async_copy(ev_h.at[b, 3, pl.ds(s * 8, 8)], evv.at[3], sem0)]
        for cp in scps:
            cp.wait()
        prev, cur = ga, gb
        bcps = [pltpu.async_copy(table.at[qkv.at[j]], prev.at[j], sem0)
                for j in range(8)]
        for cp in bcps:
            cp.wait()
        for cp in wcps:
            cp.wait()
        wcps = []
        plsc.subcore_barrier()
        for fch in range(_FE):
            acps = [pltpu.async_copy(evv.at[fch, j], table.at[ekv.at[j]],
                                     sem0, add=True) for j in range(8)]
            for cp in acps:
                cp.wait()
            plsc.subcore_barrier()
            gcps = [pltpu.async_copy(table.at[qkv.at[j]], cur.at[j], sem0)
                    for j in range(8)]
            for cp in gcps:
                cp.wait()
            for i in range(8):
                for j in range(8):
                    sl = pl.ds(j * 16, 16)
                    gouts[fch, i, sl] = cur[i, sl] - prev[i, sl]
            wcps.append(pltpu.async_copy(
                gouts.at[fch], efq_out.at[b, fch, pl.ds(s * 8, 8)], semw))
            plsc.subcore_barrier()
            prev, cur = cur, prev

    w = s * 2 + c
    pltpu.sync_copy(idxg.at[pl.ds(w * 16, 16)], idxv)
    gets = [pltpu.async_copy(gt2.at[idxv.at[0]], rowsv.at[0], sem0)]
    puts = []
    for t in range(16):
        if t < 15:
            if t >= 1:
                puts[t - 1].wait()
            gets.append(pltpu.async_copy(gt2.at[idxv.at[t + 1]],
                                         rowsv.at[(t + 1) % 2], sem0))
        gets[t].wait()
        puts.append(pltpu.async_copy(
            rowsv.at[t % 2], gg_out.at[pl.ds((w * 16 + t) * 128, 128)], semw))
    for cp in wcps:
        cp.wait()
    puts[14].wait()
    puts[15].wait()


def _sc_call(gt2, idxg2, qk_r, ek_r, ev_r):
    f32, i32 = jnp.float32, jnp.int32
    mesh = plsc.VectorSubcoreMesh(core_axis_name="c", subcore_axis_name="s",
                                  num_cores=2, num_subcores=16)
    out_type = [
        jax.ShapeDtypeStruct((_B * _NK, 128), f32),
        jax.ShapeDtypeStruct((_B, _FE, _R, _R), f32),
    ]
    scratch = [
        pltpu.VMEM((8, 128), i32),
        pltpu.VMEM((8, 128), i32),
        pltpu.VMEM((_FE, 8, 128), f32),
        pltpu.VMEM((4096,), f32),
        pltpu.VMEM((8, 128), f32),
        pltpu.VMEM((8, 128), f32),
        pltpu.VMEM((_FE, 8, 128), f32),
        pltpu.VMEM((16, 128), i32),
        pltpu.VMEM((2, 128, 128), f32),
        pltpu.SemaphoreType.DMA,
        pltpu.SemaphoreType.DMA,
        pltpu.VMEM_SHARED((_N * _N,), jnp.float32),
    ]
    return pl.kernel(_sc_body, out_type=out_type, mesh=mesh,
                     scratch_types=scratch)(gt2, idxg2, qk_r, ek_r, ev_r)


_CH = 2048
_CN = _CH // _K
_NB = 256
_NJ = _NK // _CH
_NF = _N // _NB


def _fused_body(gg_ref, c1_ref, ef2_ref, ef4_ref, ft_ref, w0c_ref, we0_ref,
                wsc_ref, out_ref, oef_ref, mtv, mzv, mefv, accy, accz):
    j = pl.program_id(0)
    b = pl.program_id(1)
    step = j * _B + b

    @pl.when(step == 0)
    def _():
        accy[...] = jnp.zeros_like(accy)
        accz[...] = jnp.zeros_like(accz)

    @pl.when(j < _NJ)
    def _stats():
        gg = gg_ref[0]
        c1 = c1_ref[0]
        ef2 = ef2_ref[0]
        dn = (((0,), (1,)), ((), ()))
        e = lax.dot_general(ef2, w0c_ref[...], dn, preferred_element_type=jnp.float32)
        c1b = jnp.reshape(jnp.broadcast_to(c1[:, None, :], (_CN, _K, 128)), (_CH, 128))
        y = gg + c1b + e
        mtv[b, pl.ds(j * _CN, _CN)] = jnp.max(jnp.reshape(y, (_CN, _K, 128)), axis=1)
        accy[0:1, :] += jnp.sum(y, axis=0, keepdims=True)
        accy[1:2, :] += jnp.sum(y * y, axis=0, keepdims=True)

        z = lax.dot_general(ef2, we0_ref[...], dn, preferred_element_type=jnp.float32)
        mzv[b, pl.ds(j * _CN, _CN)] = jnp.max(jnp.reshape(z, (_CN, _K, 16)), axis=1)
        accz[0:1, :] += jnp.sum(z, axis=0, keepdims=True)
        accz[1:2, :] += jnp.sum(z * z, axis=0, keepdims=True)

        mef = jnp.max(ef4_ref[0], axis=2)
        mefv[b, :, pl.ds(j * _CN, _CN)] = mef
        scz = lax.dot_general(mef, wsc_ref[...], dn, preferred_element_type=jnp.float32)
        accz[2:3, :] += jnp.sum(scz, axis=0, keepdims=True)
        accz[3:4, :] += jnp.sum(scz * scz, axis=0, keepdims=True)

    @pl.when(j >= _NJ)
    def _final():
        jj = j - _NJ
        ys = accy[...]
        zs = accz[...]
        cy = jnp.float32(_B * _NK)
        csc = jnp.float32(_B * _N)
        mt = mtv[b, pl.ds(jj * _NB, _NB)]
        my = ys[0:1, :] / cy
        vy = ys[1:2, :] / cy - my * my
        fts = jnp.maximum((mt - my) / jnp.sqrt(vy + _EPS), 0.0)
        out_t = jnp.maximum(ft_ref[0] + fts, 0.0)

        mzm = zs[0:1, :] / cy
        vz = zs[1:2, :] / cy - mzm * mzm
        mz = mzv[b, pl.ds(jj * _NB, _NB)]
        ftse = jnp.maximum((mz - mzm) / jnp.sqrt(vz + _EPS), 0.0)
        dn = (((0,), (1,)), ((), ()))
        mef = mefv[b, :, pl.ds(jj * _NB, _NB)]
        scz = lax.dot_general(mef, wsc_ref[...], dn, preferred_element_type=jnp.float32)
        msc = zs[2:3, :] / csc
        vsc = zs[3:4, :] / csc - msc * msc
        oef_t = jnp.maximum((scz - msc) / jnp.sqrt(vsc + _EPS) + ftse, 0.0)

        eye = (lax.broadcasted_iota(jnp.int32, (_NB, _NB), 0) ==
               lax.broadcasted_iota(jnp.int32, (_NB, _NB), 1)).astype(jnp.float32)
        dt = (((0,), (0,)), ((), ()))
        out_ref[0] = lax.dot_general(out_t, eye, dt, preferred_element_type=jnp.float32)
        oef_ref[0] = lax.dot_general(oef_t, eye, dt, preferred_element_type=jnp.float32)


def _fused_call(gg, c1t, ef2, ef4, ft_t, w0c, we0, wsc):
    f32 = jnp.float32
    sj = lambda j: jnp.minimum(j, _NJ - 1)
    fj = lambda j: jnp.maximum(j - _NJ, 0)
    return pl.pallas_call(
        _fused_body,
        grid=(_NJ + _NF, _B),
        in_specs=[
            pl.BlockSpec((1, _CH, 128), lambda j, b: (b, sj(j), 0)),
            pl.BlockSpec((1, _CN, 128), lambda j, b: (b, sj(j), 0)),
            pl.BlockSpec((1, _FE, _CH), lambda j, b: (b, 0, sj(j))),
            pl.BlockSpec((1, _FE, _CN, _K), lambda j, b: (b, 0, sj(j), 0)),
            pl.BlockSpec((1, _NB, 128), lambda j, b: (b, fj(j), 0)),
            pl.BlockSpec((128, _FE), lambda j, b: (0, 0)),
            pl.BlockSpec((16, _FE), lambda j, b: (0, 0)),
            pl.BlockSpec((16, _FE), lambda j, b: (0, 0)),
        ],
        out_specs=[
            pl.BlockSpec((1, 128, _NB), lambda j, b: (b, 0, fj(j))),
            pl.BlockSpec((1, 16, _NB), lambda j, b: (b, 0, fj(j))),
        ],
        out_shape=[
            jax.ShapeDtypeStruct((_B, 128, _N), f32),
            jax.ShapeDtypeStruct((_B, 16, _N), f32),
        ],
        scratch_shapes=[
            pltpu.VMEM((_B, _N, 128), f32),
            pltpu.VMEM((_B, _N, 16), f32),
            pltpu.VMEM((_B, _FE, _N), f32),
            pltpu.VMEM((2, 128), f32),
            pltpu.VMEM((4, 16), f32),
        ],
    )(gg, c1t, ef2, ef4, ft_t, w0c, we0, wsc)


def kernel(points, features, edge_list, edge_features, idx, W0, We0, Wsc_ef):
    del points
    idx_r = idx.reshape(_B, _R, _R)
    src_r = edge_list[:, 0, :].reshape(_B, _R, _R)
    dst_r = edge_list[:, 1, :].reshape(_B, _R, _R)
    ev_r = edge_features.reshape(_B, _FE, _R, _R)

    gt, c1t, ft_t, idxg, qk_r, ek_r = _pre_call(features, W0, idx_r, src_r, dst_r)

    gg, efq = _sc_call(gt.reshape(_B * _N, 128), idxg.reshape(_B * _R, _R),
                       qk_r, ek_r, ev_r)

    return _fused_call(
        gg.reshape(_B, _NK, 128), c1t,
        efq.reshape(_B, _FE, _NK), efq.reshape(_B, _FE, _N, _K),
        ft_t, W0[:, 2 * _D:], We0, Wsc_ef)


# speedup vs baseline: 1.0472x; 1.0472x over previous
import jax
import jax.numpy as jnp
from jax import lax
from jax.experimental import pallas as pl
from jax.experimental.pallas import tpu as pltpu
from jax.experimental.pallas import tpu_sc as plsc

_B, _D, _N, _K, _P, _FE = 4, 128, 1024, 16, 16384, 4
_NK = _N * _K
_R = 128
_EPS = 1e-5


def _pre_body(f_ref, w0_ref, idx_ref, src_ref, dst_ref,
              gt_ref, c1_ref, ftt_ref, idxg_ref, qk_ref, ek_ref):
    b = pl.program_id(0)
    x = f_ref[0]
    w0 = w0_ref[...]
    wa = w0[:, :_D]
    wb = w0[:, _D:2 * _D]
    dn = (((0,), (1,)), ((), ()))
    gt_ref[0] = lax.dot_general(x, wb, dn, preferred_element_type=jnp.float32)
    c1_ref[0] = lax.dot_general(x, wa - wb, dn, preferred_element_type=jnp.float32)
    eye = (lax.broadcasted_iota(jnp.int32, (_D, _D), 0) ==
           lax.broadcasted_iota(jnp.int32, (_D, _D), 1)).astype(jnp.float32)
    ftt_ref[0] = lax.dot_general(x, eye, dn, preferred_element_type=jnp.float32)
    idx = idx_ref[0]
    r = lax.broadcasted_iota(jnp.int32, (_R, _R), 0)
    c = lax.broadcasted_iota(jnp.int32, (_R, _R), 1)
    n = (r * _R + c) >> 4
    qk_ref[0] = (n << 10) + idx
    idxg_ref[0] = idx + b * _N
    ek_ref[0] = (src_ref[0] << 10) + dst_ref[0]


def _pre_call(features, W0, idx_r, src_r, dst_r):
    f32, i32 = jnp.float32, jnp.int32
    return pl.pallas_call(
        _pre_body,
        grid=(_B,),
        in_specs=[
            pl.BlockSpec((1, _D, _N), lambda b: (b, 0, 0)),
            pl.BlockSpec((128, 2 * _D + _FE), lambda b: (0, 0)),
            pl.BlockSpec((1, _R, _R), lambda b: (b, 0, 0)),
            pl.BlockSpec((1, _R, _R), lambda b: (b, 0, 0)),
            pl.BlockSpec((1, _R, _R), lambda b: (b, 0, 0)),
        ],
        out_specs=[
            pl.BlockSpec((1, _N, 128), lambda b: (b, 0, 0)),
            pl.BlockSpec((1, _N, 128), lambda b: (b, 0, 0)),
            pl.BlockSpec((1, _N, 128), lambda b: (b, 0, 0)),
            pl.BlockSpec((1, _R, _R), lambda b: (b, 0, 0)),
            pl.BlockSpec((1, _R, _R), lambda b: (b, 0, 0)),
            pl.BlockSpec((1, _R, _R), lambda b: (b, 0, 0)),
        ],
        out_shape=[
            jax.ShapeDtypeStruct((_B, _N, 128), f32),
            jax.ShapeDtypeStruct((_B, _N, 128), f32),
            jax.ShapeDtypeStruct((_B, _N, 128), f32),
            jax.ShapeDtypeStruct((_B, _R, _R), i32),
            jax.ShapeDtypeStruct((_B, _R, _R), i32),
            jax.ShapeDtypeStruct((_B, _R, _R), i32),
        ],
    )(features, W0, idx_r, src_r, dst_r)


def _sc_body(gt2, idxg, qk_h, ek_h, ev_h, gg_out, efq_out,
             ekv, qkv, evv, zlin, ga, gb, gouts, idxv, rowsv,
             sem0, semw, table):
    c = lax.axis_index("c")
    s = lax.axis_index("s")

    z16 = jnp.zeros((16,), jnp.float32)
    for j in range(256):
        zlin[pl.ds(j * 16, 16)] = z16

    zcps = [pltpu.async_copy(zlin, table.at[pl.ds((s * 16 + i) * 4096, 4096)],
                             sem0) for i in range(16)]
    for cp in zcps:
        cp.wait()
    plsc.subcore_barrier()

    wcps = []
    for bi in range(2):
        b = c * 2 + bi
        scps = [pltpu.async_copy(ek_h.at[b, pl.ds(s * 8, 8)], ekv, sem0),
                pltpu.async_copy(qk_h.at[b, pl.ds(s * 8, 8)], qkv, sem0),
                pltpu.async_copy(ev_h.at[b, 0, pl.ds(s * 8, 8)], evv.at[0], sem0),
                pltpu.async_copy(ev_h.at[b, 1, pl.ds(s * 8, 8)], evv.at[1], sem0),
                pltpu.async_copy(ev_h.at[b, 2, pl.ds(s * 8, 8)], evv.at[2], sem0),
                pltpu.async_copy(ev_h.at[b, 3, pl.ds(s * 8, 8)], evv.at[3], sem0)]
        for cp in scps:
            cp.wait()
        prev, cur = ga, gb
        bcps = [pltpu.async_copy(table.at[qkv.at[j]], prev.at[j], sem0)
                for j in range(8)]
        for cp in bcps:
            cp.wait()
        for cp in wcps:
            cp.wait()
        wcps = []
        plsc.subcore_barrier()
        for fch in range(_FE):
            acps = [pltpu.async_copy(evv.at[fch, j], table.at[ekv.at[j]],
                                     sem0, add=True) for j in range(8)]
            for cp in acps:
                cp.wait()
            plsc.subcore_barrier()
            gcps = [pltpu.async_copy(table.at[qkv.at[j]], cur.at[j], sem0)
                    for j in range(8)]
            for cp in gcps:
                cp.wait()
            for i in range(8):
                for j in range(8):
                    sl = pl.ds(j * 16, 16)
                    gouts[fch, i, sl] = cur[i, sl] - prev[i, sl]
            wcps.append(pltpu.async_copy(
                gouts.at[fch], efq_out.at[b, fch, pl.ds(s * 8, 8)], semw))
            plsc.subcore_barrier()
            prev, cur = cur, prev

    w = s * 2 + c
    pltpu.sync_copy(idxg.at[pl.ds(w * 16, 16)], idxv)
    gets = [pltpu.async_copy(gt2.at[idxv.at[0]], rowsv.at[0], sem0)]
    puts = []
    for t in range(16):
        if t < 15:
            if t >= 1:
                puts[t - 1].wait()
            gets.append(pltpu.async_copy(gt2.at[idxv.at[t + 1]],
                                         rowsv.at[(t + 1) % 2], sem0))
        gets[t].wait()
        puts.append(pltpu.async_copy(
            rowsv.at[t % 2], gg_out.at[pl.ds((w * 16 + t) * 128, 128)], semw))
    for cp in wcps:
        cp.wait()
    puts[14].wait()
    puts[15].wait()


def _sc_call(gt2, idxg2, qk_r, ek_r, ev_r):
    f32, i32 = jnp.float32, jnp.int32
    mesh = plsc.VectorSubcoreMesh(core_axis_name="c", subcore_axis_name="s",
                                  num_cores=2, num_subcores=16)
    out_type = [
        jax.ShapeDtypeStruct((_B * _NK, 128), f32),
        jax.ShapeDtypeStruct((_B, _FE, _R, _R), f32),
    ]
    scratch = [
        pltpu.VMEM((8, 128), i32),
        pltpu.VMEM((8, 128), i32),
        pltpu.VMEM((_FE, 8, 128), f32),
        pltpu.VMEM((4096,), f32),
        pltpu.VMEM((8, 128), f32),
        pltpu.VMEM((8, 128), f32),
        pltpu.VMEM((_FE, 8, 128), f32),
        pltpu.VMEM((16, 128), i32),
        pltpu.VMEM((2, 128, 128), f32),
        pltpu.SemaphoreType.DMA,
        pltpu.SemaphoreType.DMA,
        pltpu.VMEM_SHARED((_N * _N,), jnp.float32),
    ]
    return pl.kernel(_sc_body, out_type=out_type, mesh=mesh,
                     scratch_types=scratch)(gt2, idxg2, qk_r, ek_r, ev_r)


_CH = 2048
_CN = _CH // _K


def _stats_body(gg_ref, c1_ref, ef2_ref, ef4_ref, w0c_ref, we0_ref, wsc_ref,
                mt_ref, mz_ref, mef_ref, ys_ref, zs_ref, accy, accz):
    b = pl.program_id(0)
    j = pl.program_id(1)
    step = b * pl.num_programs(1) + j

    @pl.when(step == 0)
    def _():
        accy[...] = jnp.zeros_like(accy)
        accz[...] = jnp.zeros_like(accz)

    gg = gg_ref[0]
    c1 = c1_ref[0]
    ef2 = ef2_ref[0]
    dn = (((0,), (1,)), ((), ()))
    e = lax.dot_general(ef2, w0c_ref[...], dn, preferred_element_type=jnp.float32)
    c1b = jnp.reshape(jnp.broadcast_to(c1[:, None, :], (_CN, _K, 128)), (_CH, 128))
    y = gg + c1b + e
    mt_ref[0] = jnp.max(jnp.reshape(y, (_CN, _K, 128)), axis=1)
    accy[0:1, :] += jnp.sum(y, axis=0, keepdims=True)
    accy[1:2, :] += jnp.sum(y * y, axis=0, keepdims=True)

    z = lax.dot_general(ef2, we0_ref[...], dn, preferred_element_type=jnp.float32)
    mz_ref[0] = jnp.max(jnp.reshape(z, (_CN, _K, 16)), axis=1)
    accz[0:1, :] += jnp.sum(z, axis=0, keepdims=True)
    accz[1:2, :] += jnp.sum(z * z, axis=0, keepdims=True)

    mef = jnp.max(ef4_ref[0], axis=2)
    mef_ref[0] = mef
    scz = lax.dot_general(mef, wsc_ref[...], dn, preferred_element_type=jnp.float32)
    accz[2:3, :] += jnp.sum(scz, axis=0, keepdims=True)
    accz[3:4, :] += jnp.sum(scz * scz, axis=0, keepdims=True)

    ys_ref[...] = accy[...]
    zs_ref[...] = accz[...]


def _stats_call(gg, c1t, ef2, ef4, w0c, we0, wsc):
    f32 = jnp.float32
    nj = _NK // _CH
    return pl.pallas_call(
        _stats_body,
        grid=(_B, nj),
        in_specs=[
            pl.BlockSpec((1, _CH, 128), lambda b, j: (b, j, 0)),
            pl.BlockSpec((1, _CN, 128), lambda b, j: (b, j, 0)),
            pl.BlockSpec((1, _FE, _CH), lambda b, j: (b, 0, j)),
            pl.BlockSpec((1, _FE, _CN, _K), lambda b, j: (b, 0, j, 0)),
            pl.BlockSpec((128, _FE), lambda b, j: (0, 0)),
            pl.BlockSpec((16, _FE), lambda b, j: (0, 0)),
            pl.BlockSpec((16, _FE), lambda b, j: (0, 0)),
        ],
        out_specs=[
            pl.BlockSpec((1, _CN, 128), lambda b, j: (b, j, 0)),
            pl.BlockSpec((1, _CN, 16), lambda b, j: (b, j, 0)),
            pl.BlockSpec((1, _FE, _CN), lambda b, j: (b, 0, j)),
            pl.BlockSpec((2, 128), lambda b, j: (0, 0)),
            pl.BlockSpec((4, 16), lambda b, j: (0, 0)),
        ],
        out_shape=[
            jax.ShapeDtypeStruct((_B, _N, 128), f32),
            jax.ShapeDtypeStruct((_B, _N, 16), f32),
            jax.ShapeDtypeStruct((_B, _FE, _N), f32),
            jax.ShapeDtypeStruct((2, 128), f32),
            jax.ShapeDtypeStruct((4, 16), f32),
        ],
        scratch_shapes=[pltpu.VMEM((2, 128), f32), pltpu.VMEM((4, 16), f32)],
    )(gg, c1t, ef2, ef4, w0c, we0, wsc)


_NB = 256


def _final_body(ft_ref, mt_ref, mz_ref, mef_ref, ys_ref, zs_ref, wsc_ref,
                out_ref, oef_ref):
    ys = ys_ref[...]
    zs = zs_ref[...]
    cy = jnp.float32(_B * _NK)
    csc = jnp.float32(_B * _N)
    my = ys[0:1, :] / cy
    vy = ys[1:2, :] / cy - my * my
    fts = jnp.maximum((mt_ref[0] - my) / jnp.sqrt(vy + _EPS), 0.0)
    out_t = jnp.maximum(ft_ref[0] + fts, 0.0)

    mzm = zs[0:1, :] / cy
    vz = zs[1:2, :] / cy - mzm * mzm
    ftse = jnp.maximum((mz_ref[0] - mzm) / jnp.sqrt(vz + _EPS), 0.0)
    dn = (((0,), (1,)), ((), ()))
    scz = lax.dot_general(mef_ref[0], wsc_ref[...], dn, preferred_element_type=jnp.float32)
    msc = zs[2:3, :] / csc
    vsc = zs[3:4, :] / csc - msc * msc
    oef_t = jnp.maximum((scz - msc) / jnp.sqrt(vsc + _EPS) + ftse, 0.0)

    eye = (lax.broadcasted_iota(jnp.int32, (_NB, _NB), 0) ==
           lax.broadcasted_iota(jnp.int32, (_NB, _NB), 1)).astype(jnp.float32)
    dt = (((0,), (0,)), ((), ()))
    out_ref[0] = lax.dot_general(out_t, eye, dt, preferred_element_type=jnp.float32)
    oef_ref[0] = lax.dot_general(oef_t, eye, dt, preferred_element_type=jnp.float32)


def _final_call(ft_t, mt, mz, mef, ys, zs, wsc):
    f32 = jnp.float32
    return pl.pallas_call(
        _final_body,
        grid=(_B, _N // _NB),
        in_specs=[
            pl.BlockSpec((1, _NB, 128), lambda b, j: (b, j, 0)),
            pl.BlockSpec((1, _NB, 128), lambda b, j: (b, j, 0)),
            pl.BlockSpec((1, _NB, 16), lambda b, j: (b, j, 0)),
            pl.BlockSpec((1, _FE, _NB), lambda b, j: (b, 0, j)),
            pl.BlockSpec((2, 128), lambda b, j: (0, 0)),
            pl.BlockSpec((4, 16), lambda b, j: (0, 0)),
            pl.BlockSpec((16, _FE), lambda b, j: (0, 0)),
        ],
        out_specs=[
            pl.BlockSpec((1, 128, _NB), lambda b, j: (b, 0, j)),
            pl.BlockSpec((1, 16, _NB), lambda b, j: (b, 0, j)),
        ],
        out_shape=[
            jax.ShapeDtypeStruct((_B, 128, _N), f32),
            jax.ShapeDtypeStruct((_B, 16, _N), f32),
        ],
    )(ft_t, mt, mz, mef, ys, zs, wsc)


def kernel(points, features, edge_list, edge_features, idx, W0, We0, Wsc_ef):
    del points
    idx_r = idx.reshape(_B, _R, _R)
    src_r = edge_list[:, 0, :].reshape(_B, _R, _R)
    dst_r = edge_list[:, 1, :].reshape(_B, _R, _R)
    ev_r = edge_features.reshape(_B, _FE, _R, _R)

    gt, c1t, ft_t, idxg, qk_r, ek_r = _pre_call(features, W0, idx_r, src_r, dst_r)

    gg, efq = _sc_call(gt.reshape(_B * _N, 128), idxg.reshape(_B * _R, _R),
                       qk_r, ek_r, ev_r)

    mt, mz, mef, ys, zs = _stats_call(
        gg.reshape(_B, _NK, 128), c1t,
        efq.reshape(_B, _FE, _NK), efq.reshape(_B, _FE, _N, _K),
        W0[:, 2 * _D:], We0, Wsc_ef)

    return _final_call(ft_t, mt, mz, mef, ys, zs, Wsc_ef)
